# Initial kernel scaffold; baseline (speedup 1.0000x reference)
import jax
import jax.numpy as jnp
from jax.experimental import pallas as pl


def kernel(chain_ids, copy_ids, chain_weight, copy_weight):
    raise NotImplementedError("write your pallas kernel here")


# speedup vs baseline: 2.3490x; 2.3490x over previous
import functools

import jax
import jax.numpy as jnp
from jax import lax
from jax.experimental import pallas as pl
from jax.experimental.pallas import tpu as pltpu
from jax.experimental.pallas import tpu_sc as plsc

B0, B1 = 4096, 200
N_ROWS = B0 * B1
D = 64
N_CHAIN = 17
N_COPY = 9
N_FUSED = N_CHAIN * N_COPY
NC, NS, L = 2, 16, 16
NW = NC * NS
ROWS_PER_W = N_ROWS // NW
CHUNK = 512
N_CHUNKS = ROWS_PER_W // CHUNK
CHUNK_W = CHUNK * D


_mesh = plsc.VectorSubcoreMesh(core_axis_name="c", subcore_axis_name="s")


@functools.partial(
    pl.kernel,
    mesh=_mesh,
    out_type=jax.ShapeDtypeStruct((N_ROWS * D,), jnp.float32),
    compiler_params=pltpu.CompilerParams(needs_layout_passes=False),
    scratch_types=[
        pltpu.VMEM((N_CHAIN * D,), jnp.float32),
        pltpu.VMEM((N_COPY * D,), jnp.float32),
        pltpu.VMEM((N_FUSED * D,), jnp.float32),
        pltpu.VMEM((CHUNK,), jnp.int32),
        pltpu.VMEM((CHUNK,), jnp.int32),
        pltpu.VMEM((CHUNK_W,), jnp.float32),
        pltpu.VMEM((CHUNK_W,), jnp.float32),
        pltpu.SemaphoreType.DMA,
        pltpu.SemaphoreType.DMA,
    ],
)
def _emb_kernel(cid_hbm, pid_hbm, cw_hbm, pw_hbm, out_hbm,
                cw_v, pw_v, ftab, cid_v, pid_v, out0, out1, sem0, sem1):
    wid = lax.axis_index("s") * NC + lax.axis_index("c")
    base_row = wid * ROWS_PER_W

    pltpu.sync_copy(cw_hbm, cw_v)
    pltpu.sync_copy(pw_hbm, pw_v)

    lane = lax.iota(jnp.int32, L)
    lane_d = lane * D

    def build(i, carry):
        c = i // N_COPY
        p = i - c * N_COPY
        for d in range(D // L):
            cvals = plsc.load_gather(cw_v, [c * D + d * L + lane])
            pvals = plsc.load_gather(pw_v, [p * D + d * L + lane])
            plsc.store_scatter(ftab, [i * D + d * L + lane], cvals + pvals)
        return carry
    lax.fori_loop(0, N_FUSED, build, 0)

    def do_chunk(chunk, buf):
        row0 = base_row + chunk * CHUNK
        pltpu.sync_copy(cid_hbm.at[pl.ds(row0, CHUNK)], cid_v)
        pltpu.sync_copy(pid_hbm.at[pl.ds(row0, CHUNK)], pid_v)

        def group(g, carry):
            cid = plsc.load_gather(cid_v, [g * L + lane])
            pid = plsc.load_gather(pid_v, [g * L + lane])
            wbase = (cid * N_COPY + pid) * D
            pos_base = g * (L * D) + lane_d
            for d in range(D):
                vals = plsc.load_gather(ftab, [wbase + d])
                plsc.store_scatter(buf, [pos_base + d], vals)
            return carry
        lax.fori_loop(0, CHUNK // L, group, 0)

    def out_copy(chunk, buf, sem):
        row0 = base_row + chunk * CHUNK
        return pltpu.make_async_copy(
            buf, out_hbm.at[pl.ds(row0 * D, CHUNK_W)], sem)

    def step(s, carry):
        for b in range(2):
            chunk = s * 2 + b
            buf = (out0, out1)[b]
            sem = (sem0, sem1)[b]

            @pl.when(s > 0)
            def _wait(buf=buf, sem=sem, chunk=chunk):
                out_copy(chunk - 2, buf, sem).wait()

            do_chunk(chunk, buf)
            out_copy(chunk, buf, sem).start()
        return carry
    lax.fori_loop(0, N_CHUNKS // 2, step, 0)

    for b in range(2):
        buf = (out0, out1)[b]
        sem = (sem0, sem1)[b]
        out_copy(N_CHUNKS - 2 + b, buf, sem).wait()


def kernel(chain_ids, copy_ids, chain_weight, copy_weight):
    out = _emb_kernel(
        chain_ids.reshape(-1),
        copy_ids.reshape(-1),
        chain_weight.reshape(-1),
        copy_weight.reshape(-1),
    )
    return out.reshape(B0, B1, D)


# speedup vs baseline: 3.4492x; 1.4683x over previous
import functools

import jax
import jax.numpy as jnp
from jax import lax
from jax.experimental import pallas as pl
from jax.experimental.pallas import tpu as pltpu
from jax.experimental.pallas import tpu_sc as plsc

B0, B1 = 4096, 200
N_ROWS = B0 * B1
D = 64
N_CHAIN = 17
N_COPY = 9
N_FUSED = N_CHAIN * N_COPY
NC, NS, L = 2, 16, 16
NW = NC * NS
ROWS_PER_W = N_ROWS // NW
CHUNK = 512
N_CHUNKS = ROWS_PER_W // CHUNK
CHUNK_W = CHUNK * D


_mesh = plsc.VectorSubcoreMesh(core_axis_name="c", subcore_axis_name="s")


@functools.partial(
    pl.kernel,
    mesh=_mesh,
    out_type=jax.ShapeDtypeStruct((N_ROWS * D,), jnp.float32),
    compiler_params=pltpu.CompilerParams(needs_layout_passes=False),
    scratch_types=[
        pltpu.VMEM((N_CHAIN * D,), jnp.float32),
        pltpu.VMEM((N_COPY * D,), jnp.float32),
        pltpu.VMEM((N_FUSED * D,), jnp.float32),
        pltpu.VMEM((CHUNK,), jnp.int32),
        pltpu.VMEM((CHUNK,), jnp.int32),
        pltpu.VMEM((CHUNK_W,), jnp.float32),
        pltpu.VMEM((CHUNK_W,), jnp.float32),
        pltpu.SemaphoreType.DMA,
        pltpu.SemaphoreType.DMA,
    ],
)
def _emb_kernel(cid_hbm, pid_hbm, cw_hbm, pw_hbm, out_hbm,
                cw_v, pw_v, ftab, cid_v, pid_v, out0, out1, sem0, sem1):
    wid = lax.axis_index("s") * NC + lax.axis_index("c")
    base_row = wid * ROWS_PER_W

    pltpu.sync_copy(cw_hbm, cw_v)
    pltpu.sync_copy(pw_hbm, pw_v)

    lane = lax.iota(jnp.int32, L)
    lane_d = lane * D

    @plsc.parallel_loop(0, N_FUSED)
    def _build(i):
        c = i // N_COPY
        p = i - c * N_COPY
        for d in range(D // L):
            cvals = cw_v[pl.ds(c * D + d * L, L)]
            pvals = pw_v[pl.ds(p * D + d * L, L)]
            ftab[pl.ds(i * D + d * L, L)] = cvals + pvals

    def do_chunk(chunk, buf):
        row0 = base_row + chunk * CHUNK
        pltpu.sync_copy(cid_hbm.at[pl.ds(row0, CHUNK)], cid_v)
        pltpu.sync_copy(pid_hbm.at[pl.ds(row0, CHUNK)], pid_v)

        @plsc.parallel_loop(0, CHUNK // L)
        def _group(g):
            cid = cid_v[pl.ds(g * L, L)]
            pid = pid_v[pl.ds(g * L, L)]
            wbase = (cid * N_COPY + pid) * D
            pos_base = g * (L * D) + lane_d
            for d0 in range(0, D, 8):
                vals = [plsc.load_gather(ftab, [wbase + d]) for d in range(d0, d0 + 8)]
                for j, d in enumerate(range(d0, d0 + 8)):
                    plsc.store_scatter(buf, [pos_base + d], vals[j])

    def out_copy(chunk, buf, sem):
        row0 = base_row + chunk * CHUNK
        return pltpu.make_async_copy(
            buf, out_hbm.at[pl.ds(row0 * D, CHUNK_W)], sem)

    def step(s, carry):
        for b in range(2):
            chunk = s * 2 + b
            buf = (out0, out1)[b]
            sem = (sem0, sem1)[b]

            @pl.when(s > 0)
            def _wait(buf=buf, sem=sem, chunk=chunk):
                out_copy(chunk - 2, buf, sem).wait()

            do_chunk(chunk, buf)
            out_copy(chunk, buf, sem).start()
        return carry
    lax.fori_loop(0, N_CHUNKS // 2, step, 0)

    for b in range(2):
        buf = (out0, out1)[b]
        sem = (sem0, sem1)[b]
        out_copy(N_CHUNKS - 2 + b, buf, sem).wait()


def kernel(chain_ids, copy_ids, chain_weight, copy_weight):
    out = _emb_kernel(
        chain_ids.reshape(-1),
        copy_ids.reshape(-1),
        chain_weight.reshape(-1),
        copy_weight.reshape(-1),
    )
    return out.reshape(B0, B1, D)


# speedup vs baseline: 5.9786x; 1.7333x over previous
import functools

import jax
import jax.numpy as jnp
from jax import lax
from jax.experimental import pallas as pl
from jax.experimental.pallas import tpu as pltpu
from jax.experimental.pallas import tpu_sc as plsc

B0, B1 = 4096, 200
N_ROWS = B0 * B1
D = 64
N_CHAIN = 17
N_COPY = 9
N_FUSED = N_CHAIN * N_COPY
NC, NS, L = 2, 16, 16
NW = NC * NS
ROWS_PER_W = N_ROWS // NW
CHUNK = 512
N_CHUNKS = ROWS_PER_W // CHUNK
CHUNK_W = CHUNK * D


_mesh = plsc.VectorSubcoreMesh(core_axis_name="c", subcore_axis_name="s")


@functools.partial(
    pl.kernel,
    mesh=_mesh,
    out_type=jax.ShapeDtypeStruct((N_ROWS * D,), jnp.float32),
    compiler_params=pltpu.CompilerParams(needs_layout_passes=False),
    scratch_types=[
        pltpu.VMEM((N_CHAIN * D,), jnp.float32),
        pltpu.VMEM((N_COPY * D,), jnp.float32),
        pltpu.VMEM((N_FUSED * D,), jnp.float32),
        pltpu.VMEM((CHUNK,), jnp.int32),
        pltpu.VMEM((CHUNK,), jnp.int32),
        pltpu.VMEM((CHUNK_W,), jnp.float32),
        pltpu.VMEM((CHUNK_W,), jnp.float32),
        pltpu.SemaphoreType.DMA,
        pltpu.SemaphoreType.DMA,
    ],
)
def _emb_kernel(cid_hbm, pid_hbm, cw_hbm, pw_hbm, out_hbm,
                cw_v, pw_v, ftab, cid_v, pid_v, out0, out1, sem0, sem1):
    wid = lax.axis_index("s") * NC + lax.axis_index("c")
    base_row = wid * ROWS_PER_W

    pltpu.sync_copy(cw_hbm, cw_v)
    pltpu.sync_copy(pw_hbm, pw_v)

    lane = lax.iota(jnp.int32, L)
    lane_d = lane * D

    @plsc.parallel_loop(0, N_FUSED)
    def _build(i):
        c = i // N_COPY
        p = i - c * N_COPY
        for d in range(D // L):
            cvals = cw_v[pl.ds(c * D + d * L, L)]
            pvals = pw_v[pl.ds(p * D + d * L, L)]
            ftab[pl.ds(i * D + d * L, L)] = cvals + pvals

    def do_chunk(chunk, buf):
        row0 = base_row + chunk * CHUNK
        pltpu.sync_copy(cid_hbm.at[pl.ds(row0, CHUNK)], cid_v)
        pltpu.sync_copy(pid_hbm.at[pl.ds(row0, CHUNK)], pid_v)

        @plsc.parallel_loop(0, CHUNK // L)
        def _group(g):
            cid = cid_v[pl.ds(g * L, L)]
            pid = pid_v[pl.ds(g * L, L)]
            wbase = (cid * N_COPY + pid) * D
            pos_base = g * (L * D) + lane_d
            for d0 in range(0, D, 8):
                dds = [(lane + t) & (D - 1) for t in range(d0, d0 + 8)]
                vals = [plsc.load_gather(ftab, [wbase + dd]) for dd in dds]
                for dd, v in zip(dds, vals):
                    plsc.store_scatter(buf, [pos_base + dd], v)

    def out_copy(chunk, buf, sem):
        row0 = base_row + chunk * CHUNK
        return pltpu.make_async_copy(
            buf, out_hbm.at[pl.ds(row0 * D, CHUNK_W)], sem)

    def step(s, carry):
        for b in range(2):
            chunk = s * 2 + b
            buf = (out0, out1)[b]
            sem = (sem0, sem1)[b]

            @pl.when(s > 0)
            def _wait(buf=buf, sem=sem, chunk=chunk):
                out_copy(chunk - 2, buf, sem).wait()

            do_chunk(chunk, buf)
            out_copy(chunk, buf, sem).start()
        return carry
    lax.fori_loop(0, N_CHUNKS // 2, step, 0)

    for b in range(2):
        buf = (out0, out1)[b]
        sem = (sem0, sem1)[b]
        out_copy(N_CHUNKS - 2 + b, buf, sem).wait()


def kernel(chain_ids, copy_ids, chain_weight, copy_weight):
    out = _emb_kernel(
        chain_ids.reshape(-1),
        copy_ids.reshape(-1),
        chain_weight.reshape(-1),
        copy_weight.reshape(-1),
    )
    return out.reshape(B0, B1, D)


# speedup vs baseline: 6.3719x; 1.0658x over previous
import functools

import jax
import jax.numpy as jnp
from jax import lax
from jax.experimental import pallas as pl
from jax.experimental.pallas import tpu as pltpu
from jax.experimental.pallas import tpu_sc as plsc

B0, B1 = 4096, 200
N_ROWS = B0 * B1
D = 64
N_CHAIN = 17
N_COPY = 9
N_FUSED = N_CHAIN * N_COPY
NC, NS, L = 2, 16, 16
NW = NC * NS
ROWS_PER_W = N_ROWS // NW
CHUNK = 512
N_CHUNKS = ROWS_PER_W // CHUNK
CHUNK_W = CHUNK * D


_mesh = plsc.VectorSubcoreMesh(core_axis_name="c", subcore_axis_name="s")


@functools.partial(
    pl.kernel,
    mesh=_mesh,
    out_type=jax.ShapeDtypeStruct((N_ROWS * D,), jnp.float32),
    compiler_params=pltpu.CompilerParams(needs_layout_passes=False),
    scratch_types=[
        pltpu.VMEM((N_CHAIN * D,), jnp.float32),
        pltpu.VMEM((N_COPY * D,), jnp.float32),
        pltpu.VMEM((N_FUSED * D,), jnp.float32),
        pltpu.VMEM((CHUNK,), jnp.int32),
        pltpu.VMEM((CHUNK,), jnp.int32),
        pltpu.VMEM((CHUNK,), jnp.int32),
        pltpu.VMEM((CHUNK,), jnp.int32),
        pltpu.VMEM((CHUNK_W,), jnp.float32),
        pltpu.VMEM((CHUNK_W,), jnp.float32),
        pltpu.SemaphoreType.DMA,
        pltpu.SemaphoreType.DMA,
        pltpu.SemaphoreType.DMA,
        pltpu.SemaphoreType.DMA,
    ],
)
def _emb_kernel(cid_hbm, pid_hbm, cw_hbm, pw_hbm, out_hbm,
                cw_v, pw_v, ftab, cid0, cid1, pid0, pid1, out0, out1,
                sem0, sem1, isem0, isem1):
    wid = lax.axis_index("s") * NC + lax.axis_index("c")
    base_row = wid * ROWS_PER_W

    pltpu.sync_copy(cw_hbm, cw_v)
    pltpu.sync_copy(pw_hbm, pw_v)

    lane = lax.iota(jnp.int32, L)
    lane_d = lane * D

    @plsc.parallel_loop(0, N_FUSED)
    def _build(i):
        c = i // N_COPY
        p = i - c * N_COPY
        for d in range(D // L):
            cvals = cw_v[pl.ds(c * D + d * L, L)]
            pvals = pw_v[pl.ds(p * D + d * L, L)]
            ftab[pl.ds(i * D + d * L, L)] = cvals + pvals

    def idx_copies(chunk, cbuf, pbuf, isem):
        row0 = base_row + chunk * CHUNK
        return (pltpu.make_async_copy(cid_hbm.at[pl.ds(row0, CHUNK)], cbuf, isem),
                pltpu.make_async_copy(pid_hbm.at[pl.ds(row0, CHUNK)], pbuf, isem))

    def do_chunk(chunk, cbuf, pbuf, buf):
        @plsc.parallel_loop(0, CHUNK // L)
        def _group(g):
            cid = cbuf[pl.ds(g * L, L)]
            pid = pbuf[pl.ds(g * L, L)]
            wbase = (cid * N_COPY + pid) * D
            pos_base = g * (L * D) + lane_d
            for d0 in range(0, D, 8):
                dds = [(lane + t) & (D - 1) for t in range(d0, d0 + 8)]
                vals = [plsc.load_gather(ftab, [wbase + dd]) for dd in dds]
                for dd, v in zip(dds, vals):
                    plsc.store_scatter(buf, [pos_base + dd], v)

    def out_copy(chunk, buf, sem):
        row0 = base_row + chunk * CHUNK
        return pltpu.make_async_copy(
            buf, out_hbm.at[pl.ds(row0 * D, CHUNK_W)], sem)

    cbufs, pbufs, bufs, sems, isems = (
        (cid0, cid1), (pid0, pid1), (out0, out1), (sem0, sem1), (isem0, isem1))

    for cp in idx_copies(0, cid0, pid0, isem0):
        cp.start()

    def step(s, carry):
        for b in range(2):
            chunk = s * 2 + b

            def _prefetch(chunk=chunk, b=b):
                for cp in idx_copies(chunk + 1, cbufs[1 - b], pbufs[1 - b],
                                     isems[1 - b]):
                    cp.start()
            if b == 0:
                _prefetch()
            else:
                pl.when(s < N_CHUNKS // 2 - 1)(_prefetch)

            for cp in idx_copies(chunk, cbufs[b], pbufs[b], isems[b]):
                cp.wait()

            @pl.when(s > 0)
            def _wait(b=b, chunk=chunk):
                out_copy(chunk - 2, bufs[b], sems[b]).wait()

            do_chunk(chunk, cbufs[b], pbufs[b], bufs[b])
            out_copy(chunk, bufs[b], sems[b]).start()
        return carry
    lax.fori_loop(0, N_CHUNKS // 2, step, 0)

    for b in range(2):
        out_copy(N_CHUNKS - 2 + b, bufs[b], sems[b]).wait()


def kernel(chain_ids, copy_ids, chain_weight, copy_weight):
    out = _emb_kernel(
        chain_ids.reshape(-1),
        copy_ids.reshape(-1),
        chain_weight.reshape(-1),
        copy_weight.reshape(-1),
    )
    return out.reshape(B0, B1, D)


# speedup vs baseline: 12.0428x; 1.8900x over previous
import functools

import jax
import jax.numpy as jnp
from jax import lax
from jax.experimental import pallas as pl
from jax.experimental.pallas import tpu as pltpu
from jax.experimental.pallas import tpu_sc as plsc

B0, B1 = 4096, 200
N_ROWS = B0 * B1
D = 64
N_CHAIN = 17
N_COPY = 9
N_FUSED = N_CHAIN * N_COPY
NC, NS, L = 2, 16, 16
NW = NC * NS
OUTER_PER_W = B0 // NW
CHUNK_O = 2
CHUNK = CHUNK_O * B1
N_CHUNKS = OUTER_PER_W // CHUNK_O
CHUNK_W = CHUNK * D


_mesh = plsc.VectorSubcoreMesh(core_axis_name="c", subcore_axis_name="s")


@functools.partial(
    pl.kernel,
    mesh=_mesh,
    out_type=jax.ShapeDtypeStruct((B0, B1, D), jnp.float32),
    compiler_params=pltpu.CompilerParams(needs_layout_passes=False),
    scratch_types=[
        pltpu.VMEM((N_CHAIN * D,), jnp.float32),
        pltpu.VMEM((N_COPY * D,), jnp.float32),
        pltpu.VMEM((N_FUSED * D,), jnp.float32),
        pltpu.VMEM((CHUNK,), jnp.int32),
        pltpu.VMEM((CHUNK,), jnp.int32),
        pltpu.VMEM((CHUNK,), jnp.int32),
        pltpu.VMEM((CHUNK,), jnp.int32),
        pltpu.VMEM((CHUNK_O, B1, D), jnp.float32),
        pltpu.VMEM((CHUNK_O, B1, D), jnp.float32),
        pltpu.SemaphoreType.DMA,
        pltpu.SemaphoreType.DMA,
        pltpu.SemaphoreType.DMA,
        pltpu.SemaphoreType.DMA,
    ],
)
def _emb_kernel(cid_hbm, pid_hbm, cw_hbm, pw_hbm, out_hbm,
                cw_v, pw_v, ftab, cid0, cid1, pid0, pid1, out0, out1,
                sem0, sem1, isem0, isem1):
    wid = lax.axis_index("s") * NC + lax.axis_index("c")
    base_outer = wid * OUTER_PER_W
    base_row = base_outer * B1

    pltpu.sync_copy(cw_hbm, cw_v)
    pltpu.sync_copy(pw_hbm, pw_v)

    lane = lax.iota(jnp.int32, L)

    @plsc.parallel_loop(0, N_FUSED)
    def _build(i):
        c = i // N_COPY
        p = i - c * N_COPY
        for d in range(D // L):
            cvals = cw_v[pl.ds(c * D + d * L, L)]
            pvals = pw_v[pl.ds(p * D + d * L, L)]
            ftab[pl.ds(i * D + d * L, L)] = cvals + pvals

    def idx_copies(chunk, cbuf, pbuf, isem):
        row0 = base_row + chunk * CHUNK
        return (pltpu.make_async_copy(cid_hbm.at[pl.ds(row0, CHUNK)], cbuf, isem),
                pltpu.make_async_copy(pid_hbm.at[pl.ds(row0, CHUNK)], pbuf, isem))

    def do_chunk(chunk, cbuf, pbuf, buf):
        @plsc.parallel_loop(0, CHUNK // L)
        def _group(g):
            cid = cbuf[pl.ds(g * L, L)]
            pid = pbuf[pl.ds(g * L, L)]
            wbase = (cid * N_COPY + pid) * D
            lrow = g * L + lane
            o = lrow // B1
            j = lrow - o * B1
            for d0 in range(0, D, 8):
                dds = [(lane + t) & (D - 1) for t in range(d0, d0 + 8)]
                vals = [plsc.load_gather(ftab, [wbase + dd]) for dd in dds]
                for dd, v in zip(dds, vals):
                    plsc.store_scatter(buf, [o, j, dd], v)

    def out_copy(chunk, buf, sem):
        outer0 = base_outer + chunk * CHUNK_O
        return pltpu.make_async_copy(
            buf, out_hbm.at[pl.ds(outer0, CHUNK_O), :, :], sem)

    cbufs, pbufs, bufs, sems, isems = (
        (cid0, cid1), (pid0, pid1), (out0, out1), (sem0, sem1), (isem0, isem1))

    for cp in idx_copies(0, cid0, pid0, isem0):
        cp.start()

    def step(s, carry):
        for b in range(2):
            chunk = s * 2 + b

            def _prefetch(chunk=chunk, b=b):
                for cp in idx_copies(chunk + 1, cbufs[1 - b], pbufs[1 - b],
                                     isems[1 - b]):
                    cp.start()
            if b == 0:
                _prefetch()
            else:
                pl.when(s < N_CHUNKS // 2 - 1)(_prefetch)

            for cp in idx_copies(chunk, cbufs[b], pbufs[b], isems[b]):
                cp.wait()

            @pl.when(s > 0)
            def _wait(b=b, chunk=chunk):
                out_copy(chunk - 2, bufs[b], sems[b]).wait()

            do_chunk(chunk, cbufs[b], pbufs[b], bufs[b])
            out_copy(chunk, bufs[b], sems[b]).start()
        return carry
    lax.fori_loop(0, N_CHUNKS // 2, step, 0)

    for b in range(2):
        out_copy(N_CHUNKS - 2 + b, bufs[b], sems[b]).wait()


def kernel(chain_ids, copy_ids, chain_weight, copy_weight):
    return _emb_kernel(
        chain_ids.reshape(-1),
        copy_ids.reshape(-1),
        chain_weight.reshape(-1),
        copy_weight.reshape(-1),
    )


# speedup vs baseline: 12.4171x; 1.0311x over previous
import functools

import jax
import jax.numpy as jnp
from jax import lax
from jax.experimental import pallas as pl
from jax.experimental.pallas import tpu as pltpu
from jax.experimental.pallas import tpu_sc as plsc

B0, B1 = 4096, 200
N_ROWS = B0 * B1
D = 64
N_CHAIN = 17
N_COPY = 9
N_FUSED = N_CHAIN * N_COPY
NC, NS, L = 2, 16, 16
NW = NC * NS
OUTER_PER_W = B0 // NW
CHUNK_O = 2
CHUNK = CHUNK_O * B1
N_CHUNKS = OUTER_PER_W // CHUNK_O
CHUNK_W = CHUNK * D


_mesh = plsc.VectorSubcoreMesh(core_axis_name="c", subcore_axis_name="s")


@functools.partial(
    pl.kernel,
    mesh=_mesh,
    out_type=jax.ShapeDtypeStruct((B0, B1, D), jnp.float32),
    compiler_params=pltpu.CompilerParams(needs_layout_passes=False),
    scratch_types=[
        pltpu.VMEM((N_CHAIN * D,), jnp.float32),
        pltpu.VMEM((N_COPY * D,), jnp.float32),
        pltpu.VMEM((N_FUSED * D,), jnp.float32),
        pltpu.VMEM((CHUNK,), jnp.int32),
        pltpu.VMEM((CHUNK,), jnp.int32),
        pltpu.VMEM((CHUNK,), jnp.int32),
        pltpu.VMEM((CHUNK,), jnp.int32),
        pltpu.VMEM((CHUNK_O, B1, D), jnp.float32),
        pltpu.VMEM((CHUNK_O, B1, D), jnp.float32),
        pltpu.SemaphoreType.DMA,
        pltpu.SemaphoreType.DMA,
        pltpu.SemaphoreType.DMA,
        pltpu.SemaphoreType.DMA,
    ],
)
def _emb_kernel(cid_hbm, pid_hbm, cw_hbm, pw_hbm, out_hbm,
                cw_v, pw_v, ftab, cid0, cid1, pid0, pid1, out0, out1,
                sem0, sem1, isem0, isem1):
    wid = lax.axis_index("s") * NC + lax.axis_index("c")
    base_outer = wid * OUTER_PER_W
    base_row = base_outer * B1

    pltpu.sync_copy(cw_hbm, cw_v)
    pltpu.sync_copy(pw_hbm, pw_v)

    @plsc.parallel_loop(0, N_FUSED)
    def _build(i):
        c = i // N_COPY
        p = i - c * N_COPY
        for d in range(D // L):
            cvals = cw_v[pl.ds(c * D + d * L, L)]
            pvals = pw_v[pl.ds(p * D + d * L, L)]
            ftab[pl.ds(i * D + d * L, L)] = cvals + pvals

    def idx_copies(chunk, cbuf, pbuf, isem):
        row0 = base_row + chunk * CHUNK
        return (pltpu.make_async_copy(cid_hbm.at[pl.ds(row0, CHUNK)], cbuf, isem),
                pltpu.make_async_copy(pid_hbm.at[pl.ds(row0, CHUNK)], pbuf, isem))

    def do_chunk(chunk, cbuf, pbuf, buf):
        @plsc.parallel_loop(0, CHUNK // L)
        def _group(g):
            cid = cbuf[pl.ds(g * L, L)]
            pid = pbuf[pl.ds(g * L, L)]
            fb = (cid * N_COPY + pid) * D
            row0 = g * L
            for k in range(L):
                s_fb = fb[k]
                row = row0 + k
                o = (row >= B1).astype(jnp.int32)
                j = row - o * B1
                for q in range(D // L):
                    vals = ftab[pl.ds(s_fb + q * L, L)]
                    buf[o, j, pl.ds(q * L, L)] = vals

    def out_copy(chunk, buf, sem):
        outer0 = base_outer + chunk * CHUNK_O
        return pltpu.make_async_copy(
            buf, out_hbm.at[pl.ds(outer0, CHUNK_O), :, :], sem)

    cbufs, pbufs, bufs, sems, isems = (
        (cid0, cid1), (pid0, pid1), (out0, out1), (sem0, sem1), (isem0, isem1))

    for cp in idx_copies(0, cid0, pid0, isem0):
        cp.start()

    def step(s, carry):
        for b in range(2):
            chunk = s * 2 + b

            def _prefetch(chunk=chunk, b=b):
                for cp in idx_copies(chunk + 1, cbufs[1 - b], pbufs[1 - b],
                                     isems[1 - b]):
                    cp.start()
            if b == 0:
                _prefetch()
            else:
                pl.when(s < N_CHUNKS // 2 - 1)(_prefetch)

            for cp in idx_copies(chunk, cbufs[b], pbufs[b], isems[b]):
                cp.wait()

            @pl.when(s > 0)
            def _wait(b=b, chunk=chunk):
                out_copy(chunk - 2, bufs[b], sems[b]).wait()

            do_chunk(chunk, cbufs[b], pbufs[b], bufs[b])
            out_copy(chunk, bufs[b], sems[b]).start()
        return carry
    lax.fori_loop(0, N_CHUNKS // 2, step, 0)

    for b in range(2):
        out_copy(N_CHUNKS - 2 + b, bufs[b], sems[b]).wait()


def kernel(chain_ids, copy_ids, chain_weight, copy_weight):
    return _emb_kernel(
        chain_ids.reshape(-1),
        copy_ids.reshape(-1),
        chain_weight.reshape(-1),
        copy_weight.reshape(-1),
    )


# speedup vs baseline: 44.2601x; 3.5644x over previous
import functools

import jax
import jax.numpy as jnp
from jax import lax
from jax.experimental import pallas as pl
from jax.experimental.pallas import tpu as pltpu
from jax.experimental.pallas import tpu_sc as plsc

B0, B1 = 4096, 200
D = 64
N_CHAIN = 17
N_COPY = 9
N_FUSED = N_CHAIN * N_COPY
FSTRIDE = D + 1
NC, NS, L = 2, 16, 16
NW = NC * NS
KT = D // 8
N_SLABS = B1 * KT
SLABS_PER_W = N_SLABS // NW
N_GROUPS = B0 // L


_mesh = plsc.VectorSubcoreMesh(core_axis_name="c", subcore_axis_name="s")


@functools.partial(
    pl.kernel,
    mesh=_mesh,
    out_type=jax.ShapeDtypeStruct((B1, D, B0), jnp.float32),
    compiler_params=pltpu.CompilerParams(needs_layout_passes=False),
    scratch_types=[
        pltpu.VMEM((N_CHAIN * D,), jnp.float32),
        pltpu.VMEM((N_COPY * D,), jnp.float32),
        pltpu.VMEM((N_FUSED * FSTRIDE,), jnp.float32),
        pltpu.VMEM((B0,), jnp.int32),
        pltpu.VMEM((B0,), jnp.int32),
        pltpu.VMEM((B0,), jnp.int32),
        pltpu.VMEM((B0,), jnp.int32),
        pltpu.VMEM((1, 8, B0), jnp.float32),
        pltpu.VMEM((1, 8, B0), jnp.float32),
        pltpu.SemaphoreType.DMA,
        pltpu.SemaphoreType.DMA,
        pltpu.SemaphoreType.DMA,
        pltpu.SemaphoreType.DMA,
    ],
)
def _emb_kernel(cid_hbm, pid_hbm, cw_hbm, pw_hbm, out_hbm,
                cw_v, pw_v, ftab, cid0, cid1, pid0, pid1, out0, out1,
                sem0, sem1, isem0, isem1):
    wid = lax.axis_index("s") * NC + lax.axis_index("c")
    base_slab = wid * SLABS_PER_W

    pltpu.sync_copy(cw_hbm, cw_v)
    pltpu.sync_copy(pw_hbm, pw_v)

    @plsc.parallel_loop(0, N_FUSED)
    def _build(i):
        c = i // N_COPY
        p = i - c * N_COPY
        for d in range(D // L):
            cvals = cw_v[pl.ds(c * D + d * L, L)]
            pvals = pw_v[pl.ds(p * D + d * L, L)]
            ftab[pl.ds(i * FSTRIDE + d * L, L)] = cvals + pvals

    def idx_copies(slab, cbuf, pbuf, isem):
        j = (base_slab + slab) // KT
        row0 = j * B0
        return (pltpu.make_async_copy(cid_hbm.at[pl.ds(row0, B0)], cbuf, isem),
                pltpu.make_async_copy(pid_hbm.at[pl.ds(row0, B0)], pbuf, isem))

    def do_slab(slab, cbuf, pbuf, buf):
        kt = (base_slab + slab) % KT

        @plsc.parallel_loop(0, N_GROUPS)
        def _group(g):
            cid = cbuf[pl.ds(g * L, L)]
            pid = pbuf[pl.ds(g * L, L)]
            fidx = cid * N_COPY + pid
            wb = (fidx * D + fidx) + kt * 8
            i0 = g * L
            for ks in range(8):
                vals = plsc.load_gather(ftab, [wb + ks])
                buf[0, ks, pl.ds(i0, L)] = vals

    def out_copy(slab, buf, sem):
        s = base_slab + slab
        j = s // KT
        kt = s - j * KT
        return pltpu.make_async_copy(
            buf, out_hbm.at[pl.ds(j, 1), pl.ds(kt * 8, 8), :], sem)

    cbufs, pbufs, bufs, sems, isems = (
        (cid0, cid1), (pid0, pid1), (out0, out1), (sem0, sem1), (isem0, isem1))

    for cp in idx_copies(0, cid0, pid0, isem0):
        cp.start()

    def step(s, carry):
        for b in range(2):
            slab = s * 2 + b

            def _prefetch(slab=slab, b=b):
                for cp in idx_copies(slab + 1, cbufs[1 - b], pbufs[1 - b],
                                     isems[1 - b]):
                    cp.start()
            if b == 0:
                _prefetch()
            else:
                pl.when(s < SLABS_PER_W // 2 - 1)(_prefetch)

            for cp in idx_copies(slab, cbufs[b], pbufs[b], isems[b]):
                cp.wait()

            @pl.when(s > 0)
            def _wait(b=b, slab=slab):
                out_copy(slab - 2, bufs[b], sems[b]).wait()

            do_slab(slab, cbufs[b], pbufs[b], bufs[b])
            out_copy(slab, bufs[b], sems[b]).start()
        return carry
    lax.fori_loop(0, SLABS_PER_W // 2, step, 0)

    for b in range(2):
        out_copy(SLABS_PER_W - 2 + b, bufs[b], sems[b]).wait()


def kernel(chain_ids, copy_ids, chain_weight, copy_weight):
    out_t = _emb_kernel(
        chain_ids.T.reshape(-1),
        copy_ids.T.reshape(-1),
        chain_weight.reshape(-1),
        copy_weight.reshape(-1),
    )
    return jnp.transpose(out_t, (2, 0, 1))
